# Initial kernel scaffold; baseline (speedup 1.0000x reference)
import jax
import jax.numpy as jnp
from jax.experimental import pallas as pl


def kernel(query, memory, Wk, bk, Wv, bv, Wo, bo):
    raise NotImplementedError("write your pallas kernel here")


# speedup vs baseline: 2.8855x; 2.8855x over previous
import functools

import jax
import jax.numpy as jnp
from jax import lax
from jax.experimental import pallas as pl
from jax.experimental.pallas import tpu as pltpu
from jax.experimental.pallas import tpu_sc as plsc

B = 1024
N = 16384
D = 512
H = 8
HD = D // H
K = 64
SCALE = HD ** (-0.5)

NC = 2
NS = 16
NW = NC * NS
RPW = B // NW
LANES = 16


_TN = 2048


def _stage_a_body(mem_ref, q_ref, wkT_ref, bk_ref, wvT_ref, bv_ref,
                  k_ref, v_ref, s_ref):
    m = mem_ref[...]
    kblk = jnp.dot(m, wkT_ref[...], preferred_element_type=jnp.float32) + bk_ref[...]
    vblk = jnp.dot(m, wvT_ref[...], preferred_element_type=jnp.float32) + bv_ref[...]
    k_ref[...] = kblk
    v_ref[...] = vblk
    s_ref[...] = lax.dot_general(
        q_ref[...], kblk, (((1,), (1,)), ((), ())),
        preferred_element_type=jnp.float32) * (SCALE / H)


def _stage_a(memory, query, wkT, bk2, wvT, bv2, *, interpret=False):
    grid = (N // _TN,)
    return pl.pallas_call(
        _stage_a_body,
        grid=grid,
        in_specs=[
            pl.BlockSpec((_TN, D), lambda i: (i, 0)),
            pl.BlockSpec((B, D), lambda i: (0, 0)),
            pl.BlockSpec((D, D), lambda i: (0, 0)),
            pl.BlockSpec((1, D), lambda i: (0, 0)),
            pl.BlockSpec((D, D), lambda i: (0, 0)),
            pl.BlockSpec((1, D), lambda i: (0, 0)),
        ],
        out_specs=[
            pl.BlockSpec((_TN, D), lambda i: (i, 0)),
            pl.BlockSpec((_TN, D), lambda i: (i, 0)),
            pl.BlockSpec((B, _TN), lambda i: (0, i)),
        ],
        out_shape=[
            jax.ShapeDtypeStruct((N, D), jnp.float32),
            jax.ShapeDtypeStruct((N, D), jnp.float32),
            jax.ShapeDtypeStruct((B, N), jnp.float32),
        ],
        interpret=interpret,
    )(memory, query, wkT, bk2, wvT, bv2)


_TB = 32


def _stage_d_body(q_ref, kg_ref, vg_ref, sc_ref, idx_ref, woT_ref, bo_ref,
                  ctx_ref, aw_ref, ti_ref):
    q = q_ref[...]
    kg = kg_ref[...]
    vg = vg_ref[...]

    prod = kg * q[:, None, :]
    s = prod.reshape(_TB, K, H, HD).sum(axis=3) * SCALE

    m = s.max(axis=1, keepdims=True)
    e = jnp.exp(s - m)
    w = e / e.sum(axis=1, keepdims=True)

    wfull = jnp.broadcast_to(w[:, :, :, None], (_TB, K, H, HD)).reshape(_TB, K, D)
    ctx = (wfull * vg).sum(axis=1)
    ctx_ref[...] = jnp.dot(ctx, woT_ref[...],
                           preferred_element_type=jnp.float32) + bo_ref[...]

    aw = w.mean(axis=2)

    sc = sc_ref[...]
    idx = idx_ref[...]
    si = sc[:, :, None]
    sj = sc[:, None, :]
    ii = idx[:, :, None]
    ij = idx[:, None, :]
    beats = (sj > si) | ((sj == si) & (ij < ii))
    rank = beats.astype(jnp.int32).sum(axis=2)

    onehot = (rank[:, :, None] ==
              lax.broadcasted_iota(jnp.int32, (1, 1, K), 2))
    ti_ref[...] = (onehot.astype(jnp.int32) * idx[:, :, None]).sum(axis=1)
    aw_ref[...] = (onehot.astype(jnp.float32) * aw[:, :, None]).sum(axis=1)


def _stage_d(query, kg, vg, sel_score, sel_idx, woT, bo2, *, interpret=False):
    grid = (B // _TB,)
    return pl.pallas_call(
        _stage_d_body,
        grid=grid,
        in_specs=[
            pl.BlockSpec((_TB, D), lambda i: (i, 0)),
            pl.BlockSpec((_TB, K, D), lambda i: (i, 0, 0)),
            pl.BlockSpec((_TB, K, D), lambda i: (i, 0, 0)),
            pl.BlockSpec((_TB, K), lambda i: (i, 0)),
            pl.BlockSpec((_TB, K), lambda i: (i, 0)),
            pl.BlockSpec((D, D), lambda i: (0, 0)),
            pl.BlockSpec((1, D), lambda i: (0, 0)),
        ],
        out_specs=[
            pl.BlockSpec((_TB, D), lambda i: (i, 0)),
            pl.BlockSpec((_TB, K), lambda i: (i, 0)),
            pl.BlockSpec((_TB, K), lambda i: (i, 0)),
        ],
        out_shape=[
            jax.ShapeDtypeStruct((B, D), jnp.float32),
            jax.ShapeDtypeStruct((B, K), jnp.float32),
            jax.ShapeDtypeStruct((B, K), jnp.int32),
        ],
        interpret=interpret,
    )(query, kg, vg, sel_score, sel_idx, woT, bo2)


def _f32_key(x):
    b = plsc.bitcast(x, jnp.int32)
    return b ^ ((b >> 31) & 0x7FFFFFFF)


def _key_f32(k):
    return plsc.bitcast(k ^ ((k >> 31) & 0x7FFFFFFF), jnp.float32)


def _lane_sum(vec_i32):
    return jnp.sum(vec_i32)


def _topk_sc_body(s_hbm, idx_hbm, score_hbm,
                  row_v, hist, keya, idxa, keyb, idxb, selk, seli, selsc):
    wid = lax.axis_index("s") * NC + lax.axis_index("c")
    lane = lax.iota(jnp.int32, 16)
    zeros16 = jnp.zeros((16,), jnp.int32)
    ones16 = jnp.ones((16,), jnp.int32)

    def per_row(r, _):
        row = wid * RPW + r
        pltpu.sync_copy(s_hbm.at[row], row_v)

        def zh(i, _c):
            hist[pl.ds(i * 16, 16)] = zeros16
            return 0
        lax.fori_loop(0, 256, zh, 0, unroll=8)

        addr_base = lane + 2048

        def hbody(i, _c):
            key = _f32_key(row_v[pl.ds(i * 16, 16)])
            addr = ((key >> 24) << 4) + addr_base
            plsc.addupdate_scatter(hist, [addr], ones16)
            return 0
        lax.fori_loop(0, N // 16, hbody, 0, unroll=8)

        def dscan(i, c):
            cum, d0 = c
            d = 255 - i
            cnt = _lane_sum(hist[pl.ds(d * 16, 16)])
            newcum = cum + cnt
            hit = (newcum >= K) & (cum < K)
            return (newcum, jnp.where(hit, d, d0))
        _, d0 = lax.fori_loop(0, 256, dscan, (0, 0), unroll=4)
        d0s = d0 - 128

        def cbody(i, off):
            key = _f32_key(row_v[pl.ds(i * 16, 16)])
            msk = (key >> 24) >= d0s
            mi = msk.astype(jnp.int32)
            pos = off + plsc.cumsum(mi) - mi
            plsc.store_scatter(keya, [pos], key, mask=msk)
            plsc.store_scatter(idxa, [pos], lane + i * 16, mask=msk)
            return off + plsc.all_reduce_population_count(msk)
        offv = lax.fori_loop(0, N // 16, cbody, zeros16, unroll=4)
        m_cnt = _lane_sum(offv) // 16

        def split_body(i, c):
            offc, offs = c
            key = keya[pl.ds(i * 16, 16)]
            idx = idxa[pl.ds(i * 16, 16)]
            valid = (lane + i * 16) < m_cnt
            dig = key >> 24
            mskS = (dig > d0s) & valid
            miS = mskS.astype(jnp.int32)
            posS = offs + plsc.cumsum(miS) - miS
            plsc.store_scatter(selk, [posS], key, mask=mskS)
            plsc.store_scatter(seli, [posS], idx, mask=mskS)
            mskC = (dig == d0s) & valid
            miC = mskC.astype(jnp.int32)
            posC = offc + plsc.cumsum(miC) - miC
            plsc.store_scatter(keyb, [posC], key, mask=mskC)
            plsc.store_scatter(idxb, [posC], idx, mask=mskC)
            return (offc + plsc.all_reduce_population_count(mskC),
                    offs + plsc.all_reduce_population_count(mskS))
        offc0, nselv = lax.fori_loop(0, (m_cnt + 15) // 16, split_body,
                                     (zeros16, zeros16))
        m0 = _lane_sum(offc0) // 16

        def do_level(shift, nbits, m, nselv, src_k, src_i, dst_k, dst_i):
            nbins = 1 << nbits
            bmask = nbins - 1

            def zh2(i, _c):
                hist[pl.ds(i * 16, 16)] = zeros16
                return 0
            lax.fori_loop(0, nbins, zh2, 0)

            def hb2(i, _c):
                key = src_k[pl.ds(i * 16, 16)]
                valid = (lane + i * 16) < m
                addr = (((key >> shift) & bmask) << 4) + lane
                plsc.addupdate_scatter(hist, [addr], ones16, mask=valid)
                return 0
            nv = (m + 15) // 16
            lax.fori_loop(0, nv, hb2, 0)

            nsel = _lane_sum(nselv) // 16

            def dscan2(i, c):
                cum, dl, above = c
                d = nbins - 1 - i
                cnt = _lane_sum(hist[pl.ds(d * 16, 16)])
                newcum = cum + cnt
                hit = (newcum + nsel >= K) & (cum + nsel < K)
                return (newcum,
                        jnp.where(hit, d, dl),
                        jnp.where(hit, cum, above))
            _, dl, _above = lax.fori_loop(0, nbins, dscan2, (0, 0, 0))

            def fbody(i, c):
                offc, offs = c
                key = src_k[pl.ds(i * 16, 16)]
                idx = src_i[pl.ds(i * 16, 16)]
                valid = (lane + i * 16) < m
                dig = (key >> shift) & bmask
                mskS = (dig > dl) & valid
                miS = mskS.astype(jnp.int32)
                posS = offs + plsc.cumsum(miS) - miS
                plsc.store_scatter(selk, [posS], key, mask=mskS)
                plsc.store_scatter(seli, [posS], idx, mask=mskS)
                mskC = (dig == dl) & valid
                miC = mskC.astype(jnp.int32)
                posC = offc + plsc.cumsum(miC) - miC
                plsc.store_scatter(dst_k, [posC], key, mask=mskC)
                plsc.store_scatter(dst_i, [posC], idx, mask=mskC)
                return (offc + plsc.all_reduce_population_count(mskC),
                        offs + plsc.all_reduce_population_count(mskS))
            offc, offs = lax.fori_loop(0, nv, fbody, (zeros16, nselv))
            return _lane_sum(offc) // 16, offs

        m1, nselv = do_level(19, 5, m0, nselv, keyb, idxb, keya, idxa)
        m2, nselv = do_level(14, 5, m1, nselv, keya, idxa, keyb, idxb)
        m3, nselv = do_level(9, 5, m2, nselv, keyb, idxb, keya, idxa)
        m4, nselv = do_level(4, 5, m3, nselv, keya, idxa, keyb, idxb)
        m5, nselv = do_level(0, 4, m4, nselv, keyb, idxb, keya, idxa)

        nsel = _lane_sum(nselv) // 16
        nrem = K - nsel

        def tail(i, c):
            offs = c
            key = keya[pl.ds(i * 16, 16)]
            idx = idxa[pl.ds(i * 16, 16)]
            gpos = lane + i * 16
            msk = (gpos < nrem) & (gpos < m5)
            mi = msk.astype(jnp.int32)
            pos = offs + plsc.cumsum(mi) - mi
            plsc.store_scatter(selk, [pos], key, mask=msk)
            plsc.store_scatter(seli, [pos], idx, mask=msk)
            return offs + plsc.all_reduce_population_count(msk)
        lax.fori_loop(0, (K + 15) // 16, tail, nselv)

        def wout(i, _c):
            sl = pl.ds(i * 16, 16)
            selsc[sl] = _key_f32(selk[sl])
            return 0
        lax.fori_loop(0, K // 16, wout, 0)

        pltpu.sync_copy(seli.at[pl.ds(0, K)], idx_hbm.at[pl.ds(row * K, K)])
        pltpu.sync_copy(selsc, score_hbm.at[pl.ds(row * K, K)])
        return 0

    lax.fori_loop(0, RPW, per_row, 0)


def _topk_sc(S):
    mesh = plsc.VectorSubcoreMesh(core_axis_name="c", subcore_axis_name="s")
    f = pl.kernel(
        _topk_sc_body,
        out_type=[
            jax.ShapeDtypeStruct((B * K,), jnp.int32),
            jax.ShapeDtypeStruct((B * K,), jnp.float32),
        ],
        mesh=mesh,
        scratch_types=[
            pltpu.VMEM((N,), jnp.float32),
            pltpu.VMEM((4096,), jnp.int32),
            pltpu.VMEM((N,), jnp.int32),
            pltpu.VMEM((N,), jnp.int32),
            pltpu.VMEM((N,), jnp.int32),
            pltpu.VMEM((N,), jnp.int32),
            pltpu.VMEM((128,), jnp.int32),
            pltpu.VMEM((128,), jnp.int32),
            pltpu.VMEM((K,), jnp.float32),
        ],
        compiler_params=pltpu.CompilerParams(needs_layout_passes=False),
    )
    return f(S)


_QPW = B // NW


def _gather_sc_body(k_hbm, v_hbm, idx_hbm, kg_hbm, vg_hbm,
                    idx_v, kbuf, vbuf, semk, semv):
    wid = lax.axis_index("s") * NC + lax.axis_index("c")

    def per_q(q, _):
        row = wid * _QPW + q
        pltpu.sync_copy(idx_hbm.at[pl.ds(row * K, K)], idx_v)
        ck = pltpu.async_copy(k_hbm.at[idx_v], kbuf, semk)
        cv = pltpu.async_copy(v_hbm.at[idx_v], vbuf, semv)
        ck.wait()
        cv.wait()
        pltpu.sync_copy(kbuf, kg_hbm.at[pl.ds(row * K, K)])
        pltpu.sync_copy(vbuf, vg_hbm.at[pl.ds(row * K, K)])
        return 0

    lax.fori_loop(0, _QPW, per_q, 0)


def _gather_sc(k, v, sel_idx):
    mesh = plsc.VectorSubcoreMesh(core_axis_name="c", subcore_axis_name="s")
    f = pl.kernel(
        _gather_sc_body,
        out_type=[
            jax.ShapeDtypeStruct((B * K, D), jnp.float32),
            jax.ShapeDtypeStruct((B * K, D), jnp.float32),
        ],
        mesh=mesh,
        scratch_types=[
            pltpu.VMEM((K,), jnp.int32),
            pltpu.VMEM((K, D), jnp.float32),
            pltpu.VMEM((K, D), jnp.float32),
            pltpu.SemaphoreType.DMA,
            pltpu.SemaphoreType.DMA,
        ],
    )
    return f(k, v, sel_idx)


def kernel(query, memory, Wk, bk, Wv, bv, Wo, bo):
    wkT = Wk.T
    wvT = Wv.T
    woT = Wo.T
    bk2 = bk.reshape(1, D)
    bv2 = bv.reshape(1, D)
    bo2 = bo.reshape(1, D)
    k, v, S = _stage_a(memory, query, wkT, bk2, wvT, bv2)
    sel_idx_f, sel_score_f = _topk_sc(S)
    kg, vg = _gather_sc(k, v, sel_idx_f)
    context, attn_weights, top_idx = _stage_d(
        query, kg.reshape(B, K, D), vg.reshape(B, K, D),
        sel_score_f.reshape(B, K), sel_idx_f.reshape(B, K), woT, bo2)
    return (context, attn_weights, top_idx)
